# Initial kernel scaffold; baseline (speedup 1.0000x reference)
import functools

import jax
import jax.numpy as jnp
from jax import lax
from jax.experimental import pallas as pl
from jax.experimental.pallas import tpu as pltpu
from jax.experimental.pallas import tpu_sc as plsc

N = 10000
E = 320000
D = 128
H = 16
NC = 16

NP = 10240
NB = 1024
NPB = NP // 16

NTILES = 32
ROWS = 79
CH = 128
PT = ROWS * CH
EP = NTILES * PT
EB = 4096

_MESH = plsc.VectorSubcoreMesh(
    core_axis_name="c", subcore_axis_name="s", num_cores=2, num_subcores=16
)


def _tc_front_body(x_ref, w_ref, a_ref, h_ref, asad_ref):
    h = jnp.dot(x_ref[...], w_ref[...], preferred_element_type=jnp.float32)
    h_ref[...] = h
    asad_ref[...] = jnp.dot(h, a_ref[...], preferred_element_type=jnp.float32)


def _tc_front(xp, W1, A1):
    return pl.pallas_call(
        _tc_front_body,
        grid=(NP // NB,),
        in_specs=[
            pl.BlockSpec((NB, D), lambda i: (i, 0)),
            pl.BlockSpec((D, H), lambda i: (0, 0)),
            pl.BlockSpec((H, 2), lambda i: (0, 0)),
        ],
        out_specs=[
            pl.BlockSpec((NB, H), lambda i: (i, 0)),
            pl.BlockSpec((NB, 2), lambda i: (i, 0)),
        ],
        out_shape=[
            jax.ShapeDtypeStruct((NP, H), jnp.float32),
            jax.ShapeDtypeStruct((NP, 2), jnp.float32),
        ],
    )(xp, W1, A1)


def _combine(acc0_ref, acc1_ref, s0_ref, s1_ref, asad_ref, h_ref, b_ref):
    al = asad_ref[...]
    lg = al[:, 0:1] + al[:, 1:2]
    lg = jnp.where(lg >= 0.0, lg, 0.2 * lg)
    ex = jnp.exp(lg)
    acc = acc0_ref[...] + acc1_ref[...] + ex * h_ref[...]
    st = s0_ref[...] + s1_ref[...] + ex
    g = acc / (st + 1e-16) + b_ref[...]
    return jnp.where(g > 0.0, g, jnp.exp(g) - 1.0)


def _tc_mid_body(acc0_ref, acc1_ref, s0_ref, s1_ref, asad_ref, h_ref, b_ref,
                 w_ref, a_ref, h2_ref, asad2_ref):
    hin = _combine(acc0_ref, acc1_ref, s0_ref, s1_ref, asad_ref, h_ref, b_ref)
    h2 = jnp.dot(hin, w_ref[...], preferred_element_type=jnp.float32)
    h2_ref[...] = h2
    asad2_ref[...] = jnp.dot(h2, a_ref[...], preferred_element_type=jnp.float32)


def _tc_tail_body(acc0_ref, acc1_ref, s0_ref, s1_ref, asad_ref, h_ref, b_ref,
                  w1a_ref, w1b_ref, fb_ref, ps_ref, pd_ref):
    hf = _combine(acc0_ref, acc1_ref, s0_ref, s1_ref, asad_ref, h_ref, b_ref)
    ps_ref[...] = jnp.dot(hf, w1a_ref[...], preferred_element_type=jnp.float32)
    pd_ref[...] = (
        jnp.dot(hf, w1b_ref[...], preferred_element_type=jnp.float32)
        + fb_ref[...]
    )


def _tc_combine_call(body, extra_specs, out_specs, out_shape, args):
    return pl.pallas_call(
        body,
        grid=(NP // NB,),
        in_specs=[
            pl.BlockSpec((NB, H), lambda i: (i, 0)),
            pl.BlockSpec((NB, H), lambda i: (i, 0)),
            pl.BlockSpec((NB, 1), lambda i: (i, 0)),
            pl.BlockSpec((NB, 1), lambda i: (i, 0)),
            pl.BlockSpec((NB, 2), lambda i: (i, 0)),
            pl.BlockSpec((NB, H), lambda i: (i, 0)),
            pl.BlockSpec((1, H), lambda i: (0, 0)),
        ] + extra_specs,
        out_specs=out_specs,
        out_shape=out_shape,
    )(*args)


def _tc_out_body(z_ref, w_ref, b_ref, o_ref):
    y = jnp.dot(z_ref[...], w_ref[...], preferred_element_type=jnp.float32)
    y = y + b_ref[...]
    m = jnp.max(y, axis=1, keepdims=True)
    lse = m + jnp.log(jnp.sum(jnp.exp(y - m), axis=1, keepdims=True))
    o_ref[...] = y - lse


def _tc_out(z, fc2_W, fc2_br):
    return pl.pallas_call(
        _tc_out_body,
        grid=(EP // EB,),
        in_specs=[
            pl.BlockSpec((EB, H), lambda i: (i, 0)),
            pl.BlockSpec((H, NC), lambda i: (0, 0)),
            pl.BlockSpec((1, NC), lambda i: (0, 0)),
        ],
        out_specs=pl.BlockSpec((EB, NC), lambda i: (i, 0)),
        out_shape=jax.ShapeDtypeStruct((EP, NC), jnp.float32),
    )(z, fc2_W, fc2_br)


def _sc_agg_body(srcr, dstr, asad, htab, accp, sp,
                 src_v, dst_v, asad_v, hbuf, exbuf, zrow, zcol, acc_sh, s_sh):
    c = lax.axis_index("c")
    s = lax.axis_index("s")
    t = c * 16 + s
    zv = jnp.zeros((16,), jnp.float32)

    def zfill(i, carry):
        zrow[i] = zv
        return carry

    lax.fori_loop(0, NPB, zfill, 0)

    def zfill1(i, carry):
        zcol[pl.ds(i * 16, 16)] = zv
        return carry

    lax.fori_loop(0, NPB // 16, zfill1, 0)
    pltpu.sync_copy(zrow, acc_sh.at[pl.ds(s * NPB, NPB)])
    pltpu.sync_copy(zcol, s_sh.at[pl.ds(s * NPB, NPB)])
    pltpu.sync_copy(asad, asad_v)
    pltpu.sync_copy(srcr.at[t], src_v)
    pltpu.sync_copy(dstr.at[t], dst_v)
    plsc.subcore_barrier()

    base = t * PT

    def step(j, carry):
        pltpu.sync_copy(htab.at[src_v.at[j]], hbuf)
        for v in range(CH // 16):
            sl = pl.ds(v * 16, 16)
            sidx = src_v[j, sl]
            didx = dst_v[j, sl]
            zc = jnp.zeros((16,), jnp.int32)
            av = plsc.load_gather(asad_v, [sidx, zc])
            bv = plsc.load_gather(asad_v, [didx, zc + 1])
            e = av + bv
            e = jnp.where(e >= 0.0, e, 0.2 * e)
            ex = jnp.exp(e)
            gid = base + j * CH + v * 16 + lax.broadcasted_iota(jnp.int32, (16,), 0)
            ex = jnp.where(gid < E, ex, 0.0)
            exbuf[sl] = ex
        for k in range(CH):
            hbuf[k] = hbuf[k] * exbuf[k]
        pltpu.sync_copy(hbuf, acc_sh.at[dst_v.at[j]], add=True)
        pltpu.sync_copy(exbuf, s_sh.at[dst_v.at[j]], add=True)
        return carry

    lax.fori_loop(0, ROWS, step, 0)
    plsc.subcore_barrier()
    pltpu.sync_copy(acc_sh.at[pl.ds(s * NPB, NPB)], accp.at[c, pl.ds(s * NPB, NPB)])
    pltpu.sync_copy(s_sh.at[pl.ds(s * NPB, NPB)], sp.at[c, pl.ds(s * NPB, NPB)])


_sc_agg = functools.partial(
    pl.kernel,
    out_type=[
        jax.ShapeDtypeStruct((2, NP, H), jnp.float32),
        jax.ShapeDtypeStruct((2, NP), jnp.float32),
    ],
    mesh=_MESH,
    scratch_types=[
        pltpu.VMEM((ROWS, CH), jnp.int32),
        pltpu.VMEM((ROWS, CH), jnp.int32),
        pltpu.VMEM((NP, 2), jnp.float32),
        pltpu.VMEM((CH, H), jnp.float32),
        pltpu.VMEM((CH,), jnp.float32),
        pltpu.VMEM((NPB, H), jnp.float32),
        pltpu.VMEM((NPB,), jnp.float32),
        pltpu.VMEM_SHARED((NP, H), jnp.float32),
        pltpu.VMEM_SHARED((NP,), jnp.float32),
    ],
)(_sc_agg_body)


def _sc_mlp_body(srcr, dstr, ps_tab, pd_tab, z_out, src_v, dst_v, buf1, buf2):
    c = lax.axis_index("c")
    s = lax.axis_index("s")
    t = c * 16 + s
    pltpu.sync_copy(srcr.at[t], src_v)
    pltpu.sync_copy(dstr.at[t], dst_v)

    def step(j, carry):
        pltpu.sync_copy(ps_tab.at[src_v.at[j]], buf1)
        pltpu.sync_copy(pd_tab.at[dst_v.at[j]], buf2)
        for k in range(CH):
            buf1[k] = jnp.maximum(buf1[k] + buf2[k], 0.0)
        pltpu.sync_copy(buf1, z_out.at[pl.ds(t * PT + j * CH, CH)])
        return carry

    lax.fori_loop(0, ROWS, step, 0)


_sc_mlp = functools.partial(
    pl.kernel,
    out_type=jax.ShapeDtypeStruct((EP, H), jnp.float32),
    mesh=_MESH,
    scratch_types=[
        pltpu.VMEM((ROWS, CH), jnp.int32),
        pltpu.VMEM((ROWS, CH), jnp.int32),
        pltpu.VMEM((CH, H), jnp.float32),
        pltpu.VMEM((CH, H), jnp.float32),
    ],
)(_sc_mlp_body)


def kernel(x, edge_index, W1, a_src1, a_dst1, b1, W2, a_src2, a_dst2, b2,
           fc1_W, fc1_b, fc2_W, fc2_b):
    src = edge_index[0].astype(jnp.int32)
    dst = edge_index[1].astype(jnp.int32)
    srcp = jnp.pad(src, (0, EP - E)).reshape(NTILES, ROWS, CH)
    dstp = jnp.pad(dst, (0, EP - E)).reshape(NTILES, ROWS, CH)
    xp = jnp.pad(x, ((0, NP - N), (0, 0)))
    A1 = jnp.stack([a_src1, a_dst1], axis=1)
    A2 = jnp.stack([a_src2, a_dst2], axis=1)

    h1, asad1 = _tc_front(xp, W1, A1)
    accp1, sp1 = _sc_agg(srcp, dstp, asad1, h1)
    h2, asad2 = _tc_combine_call(
        _tc_mid_body,
        [pl.BlockSpec((H, H), lambda i: (0, 0)),
         pl.BlockSpec((H, 2), lambda i: (0, 0))],
        [pl.BlockSpec((NB, H), lambda i: (i, 0)),
         pl.BlockSpec((NB, 2), lambda i: (i, 0))],
        [jax.ShapeDtypeStruct((NP, H), jnp.float32),
         jax.ShapeDtypeStruct((NP, 2), jnp.float32)],
        (accp1[0], accp1[1], sp1[0][:, None], sp1[1][:, None], asad1, h1,
         b1[None, :], W2, A2),
    )
    accp2, sp2 = _sc_agg(srcp, dstp, asad2, h2)
    ps, pd = _tc_combine_call(
        _tc_tail_body,
        [pl.BlockSpec((H, H), lambda i: (0, 0)),
         pl.BlockSpec((H, H), lambda i: (0, 0)),
         pl.BlockSpec((1, H), lambda i: (0, 0))],
        [pl.BlockSpec((NB, H), lambda i: (i, 0)),
         pl.BlockSpec((NB, H), lambda i: (i, 0))],
        [jax.ShapeDtypeStruct((NP, H), jnp.float32),
         jax.ShapeDtypeStruct((NP, H), jnp.float32)],
        (accp2[0], accp2[1], sp2[0][:, None], sp2[1][:, None], asad2, h2,
         b2[None, :], fc1_W[:H], fc1_W[H:], fc1_b[None, :]),
    )
    z = _sc_mlp(srcp, dstp, ps, pd)
    out = _tc_out(z, fc2_W, fc2_b[None, :])
    return out[:E]


# speedup vs baseline: 24.4296x; 24.4296x over previous
import functools

import jax
import jax.numpy as jnp
from jax import lax
from jax.experimental import pallas as pl
from jax.experimental.pallas import tpu as pltpu
from jax.experimental.pallas import tpu_sc as plsc

N = 10000
E = 320000
D = 128
H = 16
NC = 16

NP = 10240
NB = 1024
NPB = NP // 16

NTILES = 32
ROWS = 79
CH = 128
PT = ROWS * CH
EP = NTILES * PT
EB = 4096

_MESH = plsc.VectorSubcoreMesh(
    core_axis_name="c", subcore_axis_name="s", num_cores=2, num_subcores=16
)
_SC_PARAMS = pltpu.CompilerParams(
    needs_layout_passes=False, use_tc_tiling_on_sc=False
)


def _tc_front_body(x_ref, w_ref, a_ref, h_ref, asad_ref):
    h = jnp.dot(x_ref[...], w_ref[...], preferred_element_type=jnp.float32)
    h_ref[...] = h
    asad_ref[...] = jnp.dot(h, a_ref[...], preferred_element_type=jnp.float32)


def _tc_front(xp, W1, A1):
    return pl.pallas_call(
        _tc_front_body,
        grid=(NP // NB,),
        in_specs=[
            pl.BlockSpec((NB, D), lambda i: (i, 0)),
            pl.BlockSpec((D, H), lambda i: (0, 0)),
            pl.BlockSpec((H, 2), lambda i: (0, 0)),
        ],
        out_specs=[
            pl.BlockSpec((NB, H), lambda i: (i, 0)),
            pl.BlockSpec((NB, 2), lambda i: (i, 0)),
        ],
        out_shape=[
            jax.ShapeDtypeStruct((NP, H), jnp.float32),
            jax.ShapeDtypeStruct((NP, 2), jnp.float32),
        ],
    )(xp, W1, A1)


def _combine(acc0_ref, acc1_ref, s0_ref, s1_ref, asad_ref, h_ref, b_ref):
    al = asad_ref[...]
    lg = al[:, 0:1] + al[:, 1:2]
    lg = jnp.where(lg >= 0.0, lg, 0.2 * lg)
    ex = jnp.exp(lg)
    acc = acc0_ref[...] + acc1_ref[...] + ex * h_ref[...]
    st = s0_ref[...] + s1_ref[...] + ex
    g = acc / (st + 1e-16) + b_ref[...]
    return jnp.where(g > 0.0, g, jnp.exp(g) - 1.0)


def _tc_mid_body(acc0_ref, acc1_ref, s0_ref, s1_ref, asad_ref, h_ref, b_ref,
                 w_ref, a_ref, h2_ref, asad2_ref):
    hin = _combine(acc0_ref, acc1_ref, s0_ref, s1_ref, asad_ref, h_ref, b_ref)
    h2 = jnp.dot(hin, w_ref[...], preferred_element_type=jnp.float32)
    h2_ref[...] = h2
    asad2_ref[...] = jnp.dot(h2, a_ref[...], preferred_element_type=jnp.float32)


def _tc_tail_body(acc0_ref, acc1_ref, s0_ref, s1_ref, asad_ref, h_ref, b_ref,
                  w1a_ref, w1b_ref, fb_ref, ps_ref, pd_ref):
    hf = _combine(acc0_ref, acc1_ref, s0_ref, s1_ref, asad_ref, h_ref, b_ref)
    ps_ref[...] = jnp.dot(hf, w1a_ref[...], preferred_element_type=jnp.float32)
    pd_ref[...] = (
        jnp.dot(hf, w1b_ref[...], preferred_element_type=jnp.float32)
        + fb_ref[...]
    )


def _tc_combine_call(body, extra_specs, out_specs, out_shape, args):
    return pl.pallas_call(
        body,
        grid=(NP // NB,),
        in_specs=[
            pl.BlockSpec((NB, H), lambda i: (i, 0)),
            pl.BlockSpec((NB, H), lambda i: (i, 0)),
            pl.BlockSpec((NB, 1), lambda i: (i, 0)),
            pl.BlockSpec((NB, 1), lambda i: (i, 0)),
            pl.BlockSpec((NB, 2), lambda i: (i, 0)),
            pl.BlockSpec((NB, H), lambda i: (i, 0)),
            pl.BlockSpec((1, H), lambda i: (0, 0)),
        ] + extra_specs,
        out_specs=out_specs,
        out_shape=out_shape,
    )(*args)


def _tc_out_body(z_ref, w_ref, b_ref, o_ref):
    y = jnp.dot(z_ref[...], w_ref[...], preferred_element_type=jnp.float32)
    y = y + b_ref[...]
    m = jnp.max(y, axis=1, keepdims=True)
    lse = m + jnp.log(jnp.sum(jnp.exp(y - m), axis=1, keepdims=True))
    o_ref[...] = y - lse


def _tc_out(z, fc2_W, fc2_br):
    return pl.pallas_call(
        _tc_out_body,
        grid=(EP // EB,),
        in_specs=[
            pl.BlockSpec((EB, H), lambda i: (i, 0)),
            pl.BlockSpec((H, NC), lambda i: (0, 0)),
            pl.BlockSpec((1, NC), lambda i: (0, 0)),
        ],
        out_specs=pl.BlockSpec((EB, NC), lambda i: (i, 0)),
        out_shape=jax.ShapeDtypeStruct((EP, NC), jnp.float32),
    )(z, fc2_W, fc2_br)


def _sc_agg_body(srcr, dstr, a_s, a_d, htab, accp, sp,
                 src_v, dst_v, as_v, ad_v, hbuf, exbuf, zrow, zcol, acc_sh, s_sh):
    c = lax.axis_index("c")
    s = lax.axis_index("s")
    t = c * 16 + s
    zv = jnp.zeros((16,), jnp.float32)

    def zfill(i, carry):
        zrow[i] = zv
        return carry

    lax.fori_loop(0, NPB, zfill, 0)

    def zfill1(i, carry):
        zcol[pl.ds(i * 16, 16)] = zv
        return carry

    lax.fori_loop(0, NPB // 16, zfill1, 0)
    pltpu.sync_copy(zrow, acc_sh.at[pl.ds(s * NPB, NPB)])
    pltpu.sync_copy(zcol, s_sh.at[pl.ds(s * NPB, NPB)])
    pltpu.sync_copy(a_s, as_v)
    pltpu.sync_copy(a_d, ad_v)
    pltpu.sync_copy(srcr.at[t], src_v)
    pltpu.sync_copy(dstr.at[t], dst_v)
    plsc.subcore_barrier()

    base = t * PT

    def step(j, carry):
        pltpu.sync_copy(htab.at[src_v.at[j]], hbuf)
        for v in range(CH // 16):
            sl = pl.ds(v * 16, 16)
            sidx = src_v[j, sl]
            didx = dst_v[j, sl]
            av = plsc.load_gather(as_v, [sidx])
            bv = plsc.load_gather(ad_v, [didx])
            e = av + bv
            e = jnp.where(e >= 0.0, e, 0.2 * e)
            ex = jnp.exp(e)
            gid = base + j * CH + v * 16 + lax.broadcasted_iota(jnp.int32, (16,), 0)
            ex = jnp.where(gid < E, ex, 0.0)
            exbuf[sl] = ex
        for v in range(CH // 16):
            exv = exbuf[pl.ds(v * 16, 16)]
            for l in range(16):
                k = v * 16 + l
                hbuf[k] = hbuf[k] * exv[l]
        pltpu.sync_copy(hbuf, acc_sh.at[dst_v.at[j]], add=True)
        pltpu.sync_copy(exbuf, s_sh.at[dst_v.at[j]], add=True)
        return carry

    lax.fori_loop(0, ROWS, step, 0)
    plsc.subcore_barrier()
    pltpu.sync_copy(acc_sh.at[pl.ds(s * NPB, NPB)], accp.at[c, pl.ds(s * NPB, NPB)])
    pltpu.sync_copy(s_sh.at[pl.ds(s * NPB, NPB)], sp.at[c, pl.ds(s * NPB, NPB)])


_sc_agg = functools.partial(
    pl.kernel,
    out_type=[
        jax.ShapeDtypeStruct((2, NP, H), jnp.float32),
        jax.ShapeDtypeStruct((2, NP), jnp.float32),
    ],
    mesh=_MESH,
    compiler_params=_SC_PARAMS,
    scratch_types=[
        pltpu.VMEM((ROWS, CH), jnp.int32),
        pltpu.VMEM((ROWS, CH), jnp.int32),
        pltpu.VMEM((NP,), jnp.float32),
        pltpu.VMEM((NP,), jnp.float32),
        pltpu.VMEM((CH, H), jnp.float32),
        pltpu.VMEM((CH,), jnp.float32),
        pltpu.VMEM((NPB, H), jnp.float32),
        pltpu.VMEM((NPB,), jnp.float32),
        pltpu.VMEM_SHARED((NP, H), jnp.float32),
        pltpu.VMEM_SHARED((NP,), jnp.float32),
    ],
)(_sc_agg_body)


def _sc_mlp_body(srcr, dstr, ps_tab, pd_tab, z_out, src_v, dst_v, buf1, buf2):
    c = lax.axis_index("c")
    s = lax.axis_index("s")
    t = c * 16 + s
    pltpu.sync_copy(srcr.at[t], src_v)
    pltpu.sync_copy(dstr.at[t], dst_v)

    def step(j, carry):
        pltpu.sync_copy(ps_tab.at[src_v.at[j]], buf1)
        pltpu.sync_copy(pd_tab.at[dst_v.at[j]], buf2)
        for k in range(CH):
            buf1[k] = jnp.maximum(buf1[k] + buf2[k], 0.0)
        pltpu.sync_copy(buf1, z_out.at[pl.ds(t * PT + j * CH, CH)])
        return carry

    lax.fori_loop(0, ROWS, step, 0)


_sc_mlp = functools.partial(
    pl.kernel,
    out_type=jax.ShapeDtypeStruct((EP, H), jnp.float32),
    mesh=_MESH,
    compiler_params=_SC_PARAMS,
    scratch_types=[
        pltpu.VMEM((ROWS, CH), jnp.int32),
        pltpu.VMEM((ROWS, CH), jnp.int32),
        pltpu.VMEM((CH, H), jnp.float32),
        pltpu.VMEM((CH, H), jnp.float32),
    ],
)(_sc_mlp_body)


def kernel(x, edge_index, W1, a_src1, a_dst1, b1, W2, a_src2, a_dst2, b2,
           fc1_W, fc1_b, fc2_W, fc2_b):
    src = edge_index[0].astype(jnp.int32)
    dst = edge_index[1].astype(jnp.int32)
    srcp = jnp.pad(src, (0, EP - E)).reshape(NTILES, ROWS, CH)
    dstp = jnp.pad(dst, (0, EP - E)).reshape(NTILES, ROWS, CH)
    xp = jnp.pad(x, ((0, NP - N), (0, 0)))
    A1 = jnp.stack([a_src1, a_dst1], axis=1)
    A2 = jnp.stack([a_src2, a_dst2], axis=1)

    h1, asad1 = _tc_front(xp, W1, A1)
    accp1, sp1 = _sc_agg(srcp, dstp, asad1[:, 0], asad1[:, 1], h1)
    h2, asad2 = _tc_combine_call(
        _tc_mid_body,
        [pl.BlockSpec((H, H), lambda i: (0, 0)),
         pl.BlockSpec((H, 2), lambda i: (0, 0))],
        [pl.BlockSpec((NB, H), lambda i: (i, 0)),
         pl.BlockSpec((NB, 2), lambda i: (i, 0))],
        [jax.ShapeDtypeStruct((NP, H), jnp.float32),
         jax.ShapeDtypeStruct((NP, 2), jnp.float32)],
        (accp1[0], accp1[1], sp1[0][:, None], sp1[1][:, None], asad1, h1,
         b1[None, :], W2, A2),
    )
    accp2, sp2 = _sc_agg(srcp, dstp, asad2[:, 0], asad2[:, 1], h2)
    ps, pd = _tc_combine_call(
        _tc_tail_body,
        [pl.BlockSpec((H, H), lambda i: (0, 0)),
         pl.BlockSpec((H, H), lambda i: (0, 0)),
         pl.BlockSpec((1, H), lambda i: (0, 0))],
        [pl.BlockSpec((NB, H), lambda i: (i, 0)),
         pl.BlockSpec((NB, H), lambda i: (i, 0))],
        [jax.ShapeDtypeStruct((NP, H), jnp.float32),
         jax.ShapeDtypeStruct((NP, H), jnp.float32)],
        (accp2[0], accp2[1], sp2[0][:, None], sp2[1][:, None], asad2, h2,
         b2[None, :], fc1_W[:H], fc1_W[H:], fc1_b[None, :]),
    )
    z = _sc_mlp(srcp, dstp, ps, pd)
    out = _tc_out(z, fc2_W, fc2_b[None, :])
    return out[:E]


# speedup vs baseline: 31.1596x; 1.2755x over previous
import functools

import jax
import jax.numpy as jnp
from jax import lax
from jax.experimental import pallas as pl
from jax.experimental.pallas import tpu as pltpu
from jax.experimental.pallas import tpu_sc as plsc

N = 10000
E = 320000
D = 128
H = 16
NC = 16

NP = 10240
NB = 1024
NPB = NP // 16

NTILES = 32
ROWS = 81
CH = 128
PT = ROWS * CH
EP = NTILES * PT
EB = 3200

_MESH = plsc.VectorSubcoreMesh(
    core_axis_name="c", subcore_axis_name="s", num_cores=2, num_subcores=16
)
_SC_PARAMS = pltpu.CompilerParams(
    needs_layout_passes=False, use_tc_tiling_on_sc=False
)


def _tc_front_body(x_ref, w_ref, a_ref, h_ref, asad_ref):
    h = jnp.dot(x_ref[...], w_ref[...], preferred_element_type=jnp.float32)
    h_ref[...] = h
    asad_ref[...] = jnp.dot(h, a_ref[...], preferred_element_type=jnp.float32)


def _tc_front(xp, W1, A1):
    return pl.pallas_call(
        _tc_front_body,
        grid=(NP // NB,),
        in_specs=[
            pl.BlockSpec((NB, D), lambda i: (i, 0)),
            pl.BlockSpec((D, H), lambda i: (0, 0)),
            pl.BlockSpec((H, 2), lambda i: (0, 0)),
        ],
        out_specs=[
            pl.BlockSpec((NB, H), lambda i: (i, 0)),
            pl.BlockSpec((NB, 2), lambda i: (i, 0)),
        ],
        out_shape=[
            jax.ShapeDtypeStruct((NP, H), jnp.float32),
            jax.ShapeDtypeStruct((NP, 2), jnp.float32),
        ],
    )(xp, W1, A1)


def _combine(acc0_ref, acc1_ref, s0_ref, s1_ref, asad_ref, h_ref, b_ref):
    al = asad_ref[...]
    lg = al[:, 0:1] + al[:, 1:2]
    lg = jnp.where(lg >= 0.0, lg, 0.2 * lg)
    ex = jnp.exp(lg)
    acc = acc0_ref[...] + acc1_ref[...] + ex * h_ref[...]
    st = s0_ref[...] + s1_ref[...] + ex
    g = acc / (st + 1e-16) + b_ref[...]
    return jnp.where(g > 0.0, g, jnp.exp(g) - 1.0)


def _tc_mid_body(acc0_ref, acc1_ref, s0_ref, s1_ref, asad_ref, h_ref, b_ref,
                 w_ref, a_ref, h2_ref, asad2_ref):
    hin = _combine(acc0_ref, acc1_ref, s0_ref, s1_ref, asad_ref, h_ref, b_ref)
    h2 = jnp.dot(hin, w_ref[...], preferred_element_type=jnp.float32)
    h2_ref[...] = h2
    asad2_ref[...] = jnp.dot(h2, a_ref[...], preferred_element_type=jnp.float32)


def _tc_tail_body(acc0_ref, acc1_ref, s0_ref, s1_ref, asad_ref, h_ref, b_ref,
                  w1a_ref, w1b_ref, fb_ref, ps_ref, pd_ref):
    hf = _combine(acc0_ref, acc1_ref, s0_ref, s1_ref, asad_ref, h_ref, b_ref)
    ps_ref[...] = jnp.dot(hf, w1a_ref[...], preferred_element_type=jnp.float32)
    pd_ref[...] = (
        jnp.dot(hf, w1b_ref[...], preferred_element_type=jnp.float32)
        + fb_ref[...]
    )


def _tc_combine_call(body, extra_specs, out_specs, out_shape, args):
    return pl.pallas_call(
        body,
        grid=(NP // NB,),
        in_specs=[
            pl.BlockSpec((NB, H), lambda i: (i, 0)),
            pl.BlockSpec((NB, H), lambda i: (i, 0)),
            pl.BlockSpec((NB, 1), lambda i: (i, 0)),
            pl.BlockSpec((NB, 1), lambda i: (i, 0)),
            pl.BlockSpec((NB, 2), lambda i: (i, 0)),
            pl.BlockSpec((NB, H), lambda i: (i, 0)),
            pl.BlockSpec((1, H), lambda i: (0, 0)),
        ] + extra_specs,
        out_specs=out_specs,
        out_shape=out_shape,
    )(*args)


def _tc_out_body(z_ref, w_ref, b_ref, o_ref):
    y = jnp.dot(z_ref[...], w_ref[...], preferred_element_type=jnp.float32)
    y = y + b_ref[...]
    m = jnp.max(y, axis=1, keepdims=True)
    lse = m + jnp.log(jnp.sum(jnp.exp(y - m), axis=1, keepdims=True))
    o_ref[...] = y - lse


def _tc_out(z, fc2_W, fc2_br):
    return pl.pallas_call(
        _tc_out_body,
        grid=(E // EB,),
        in_specs=[
            pl.BlockSpec((EB, H), lambda i: (i, 0)),
            pl.BlockSpec((H, NC), lambda i: (0, 0)),
            pl.BlockSpec((1, NC), lambda i: (0, 0)),
        ],
        out_specs=pl.BlockSpec((EB, NC), lambda i: (i, 0)),
        out_shape=jax.ShapeDtypeStruct((E, NC), jnp.float32),
    )(z, fc2_W, fc2_br)


def _sc_agg_body(srcr, dstr, a_s, a_d, htab, accp, sp,
                 src_v, dst_v, as_v, ad_v,
                 hb0, hb1, hb2, exb0, exb1, exb2,
                 zrow, zcol, acc_sh, s_sh,
                 gs0, gs1, gs2, ss0, ss1, ss2):
    c = lax.axis_index("c")
    s = lax.axis_index("s")
    t = c * 16 + s
    zv = jnp.zeros((16,), jnp.float32)
    hbufs = (hb0, hb1, hb2)
    exbufs = (exb0, exb1, exb2)
    gsems = (gs0, gs1, gs2)
    ssems = (ss0, ss1, ss2)

    def zfill(i, carry):
        zrow[i] = zv
        return carry

    lax.fori_loop(0, NPB, zfill, 0)

    def zfill1(i, carry):
        zcol[pl.ds(i * 16, 16)] = zv
        return carry

    lax.fori_loop(0, NPB // 16, zfill1, 0)
    pltpu.sync_copy(zrow, acc_sh.at[pl.ds(s * NPB, NPB)])
    pltpu.sync_copy(zcol, s_sh.at[pl.ds(s * NPB, NPB)])
    pltpu.sync_copy(a_s, as_v)
    pltpu.sync_copy(a_d, ad_v)
    pltpu.sync_copy(srcr.at[t], src_v)
    pltpu.sync_copy(dstr.at[t], dst_v)
    plsc.subcore_barrier()

    base = t * PT

    def gissue(j, b):
        pltpu.async_copy(htab.at[src_v.at[j]], hbufs[b], gsems[b])

    def swait(b):
        pltpu.make_async_copy(hbufs[b], acc_sh.at[dst_v.at[0]], ssems[b]).wait()
        pltpu.make_async_copy(exbufs[b], s_sh.at[dst_v.at[0]], ssems[b]).wait()

    def section(j, b):
        hb = hbufs[b]
        exb = exbufs[b]
        pltpu.make_async_copy(htab.at[src_v.at[j]], hb, gsems[b]).wait()
        for v in range(CH // 16):
            sl = pl.ds(v * 16, 16)
            sidx = src_v[j, sl]
            didx = dst_v[j, sl]
            av = plsc.load_gather(as_v, [sidx])
            bv = plsc.load_gather(ad_v, [didx])
            e = av + bv
            e = jnp.where(e >= 0.0, e, 0.2 * e)
            ex = jnp.exp(e)
            gid = base + j * CH + v * 16 + lax.broadcasted_iota(jnp.int32, (16,), 0)
            ex = jnp.where(gid < E, ex, 0.0)
            exb[sl] = ex
            for l in range(16):
                k = v * 16 + l
                hb[k] = hb[k] * ex[l]
        pltpu.async_copy(hb, acc_sh.at[dst_v.at[j]], ssems[b], add=True)
        pltpu.async_copy(exb, s_sh.at[dst_v.at[j]], ssems[b], add=True)
        nb = (b + 2) % 3

        @pl.when(j < ROWS - 2)
        def _():
            @pl.when(j >= 1)
            def _():
                swait(nb)

            gissue(j + 2, nb)

    gissue(0, 0)
    gissue(1, 1)

    def gbody(g, carry):
        for k in range(3):
            section(3 * g + k, k)
        return carry

    lax.fori_loop(0, ROWS // 3, gbody, 0)
    swait(0)
    swait(1)
    swait(2)
    plsc.subcore_barrier()
    pltpu.sync_copy(acc_sh.at[pl.ds(s * NPB, NPB)], accp.at[c, pl.ds(s * NPB, NPB)])
    pltpu.sync_copy(s_sh.at[pl.ds(s * NPB, NPB)], sp.at[c, pl.ds(s * NPB, NPB)])


_sc_agg = functools.partial(
    pl.kernel,
    out_type=[
        jax.ShapeDtypeStruct((2, NP, H), jnp.float32),
        jax.ShapeDtypeStruct((2, NP), jnp.float32),
    ],
    mesh=_MESH,
    compiler_params=_SC_PARAMS,
    scratch_types=[
        pltpu.VMEM((ROWS, CH), jnp.int32),
        pltpu.VMEM((ROWS, CH), jnp.int32),
        pltpu.VMEM((NP,), jnp.float32),
        pltpu.VMEM((NP,), jnp.float32),
        pltpu.VMEM((CH, H), jnp.float32),
        pltpu.VMEM((CH, H), jnp.float32),
        pltpu.VMEM((CH, H), jnp.float32),
        pltpu.VMEM((CH,), jnp.float32),
        pltpu.VMEM((CH,), jnp.float32),
        pltpu.VMEM((CH,), jnp.float32),
        pltpu.VMEM((NPB, H), jnp.float32),
        pltpu.VMEM((NPB,), jnp.float32),
        pltpu.VMEM_SHARED((NP, H), jnp.float32),
        pltpu.VMEM_SHARED((NP,), jnp.float32),
        pltpu.SemaphoreType.DMA,
        pltpu.SemaphoreType.DMA,
        pltpu.SemaphoreType.DMA,
        pltpu.SemaphoreType.DMA,
        pltpu.SemaphoreType.DMA,
        pltpu.SemaphoreType.DMA,
    ],
)(_sc_agg_body)


def _sc_mlp_body(srcr, dstr, ps_tab, pd_tab, z_out,
                 src_v, dst_v, pa0, pa1, pa2, pb0, pb1, pb2,
                 gs0, gs1, gs2, ws0, ws1, ws2):
    c = lax.axis_index("c")
    s = lax.axis_index("s")
    t = c * 16 + s
    pas = (pa0, pa1, pa2)
    pbs = (pb0, pb1, pb2)
    gsems = (gs0, gs1, gs2)
    wsems = (ws0, ws1, ws2)
    pltpu.sync_copy(srcr.at[t], src_v)
    pltpu.sync_copy(dstr.at[t], dst_v)

    def gissue(j, b):
        pltpu.async_copy(ps_tab.at[src_v.at[j]], pas[b], gsems[b])
        pltpu.async_copy(pd_tab.at[dst_v.at[j]], pbs[b], gsems[b])

    def wwait(b):
        pltpu.make_async_copy(pas[b], z_out.at[pl.ds(t * PT, CH)], wsems[b]).wait()

    def section(j, b):
        pa = pas[b]
        pb = pbs[b]
        pltpu.make_async_copy(ps_tab.at[src_v.at[j]], pa, gsems[b]).wait()
        pltpu.make_async_copy(pd_tab.at[dst_v.at[j]], pb, gsems[b]).wait()
        for k in range(CH):
            pa[k] = jnp.maximum(pa[k] + pb[k], 0.0)
        pltpu.async_copy(pa, z_out.at[pl.ds(t * PT + j * CH, CH)], wsems[b])
        nb = (b + 2) % 3

        @pl.when(j < ROWS - 2)
        def _():
            @pl.when(j >= 1)
            def _():
                wwait(nb)

            gissue(j + 2, nb)

    gissue(0, 0)
    gissue(1, 1)

    def gbody(g, carry):
        for k in range(3):
            section(3 * g + k, k)
        return carry

    lax.fori_loop(0, ROWS // 3, gbody, 0)
    wwait(0)
    wwait(1)
    wwait(2)


_sc_mlp = functools.partial(
    pl.kernel,
    out_type=jax.ShapeDtypeStruct((EP, H), jnp.float32),
    mesh=_MESH,
    compiler_params=_SC_PARAMS,
    scratch_types=[
        pltpu.VMEM((ROWS, CH), jnp.int32),
        pltpu.VMEM((ROWS, CH), jnp.int32),
        pltpu.VMEM((CH, H), jnp.float32),
        pltpu.VMEM((CH, H), jnp.float32),
        pltpu.VMEM((CH, H), jnp.float32),
        pltpu.VMEM((CH, H), jnp.float32),
        pltpu.VMEM((CH, H), jnp.float32),
        pltpu.VMEM((CH, H), jnp.float32),
        pltpu.SemaphoreType.DMA,
        pltpu.SemaphoreType.DMA,
        pltpu.SemaphoreType.DMA,
        pltpu.SemaphoreType.DMA,
        pltpu.SemaphoreType.DMA,
        pltpu.SemaphoreType.DMA,
    ],
)(_sc_mlp_body)


def kernel(x, edge_index, W1, a_src1, a_dst1, b1, W2, a_src2, a_dst2, b2,
           fc1_W, fc1_b, fc2_W, fc2_b):
    src = edge_index[0].astype(jnp.int32)
    dst = edge_index[1].astype(jnp.int32)
    srcp = jnp.pad(src, (0, EP - E)).reshape(NTILES, ROWS, CH)
    dstp = jnp.pad(dst, (0, EP - E)).reshape(NTILES, ROWS, CH)
    xp = jnp.pad(x, ((0, NP - N), (0, 0)))
    A1 = jnp.stack([a_src1, a_dst1], axis=1)
    A2 = jnp.stack([a_src2, a_dst2], axis=1)

    h1, asad1 = _tc_front(xp, W1, A1)
    accp1, sp1 = _sc_agg(srcp, dstp, asad1[:, 0], asad1[:, 1], h1)
    h2, asad2 = _tc_combine_call(
        _tc_mid_body,
        [pl.BlockSpec((H, H), lambda i: (0, 0)),
         pl.BlockSpec((H, 2), lambda i: (0, 0))],
        [pl.BlockSpec((NB, H), lambda i: (i, 0)),
         pl.BlockSpec((NB, 2), lambda i: (i, 0))],
        [jax.ShapeDtypeStruct((NP, H), jnp.float32),
         jax.ShapeDtypeStruct((NP, 2), jnp.float32)],
        (accp1[0], accp1[1], sp1[0][:, None], sp1[1][:, None], asad1, h1,
         b1[None, :], W2, A2),
    )
    accp2, sp2 = _sc_agg(srcp, dstp, asad2[:, 0], asad2[:, 1], h2)
    ps, pd = _tc_combine_call(
        _tc_tail_body,
        [pl.BlockSpec((H, H), lambda i: (0, 0)),
         pl.BlockSpec((H, H), lambda i: (0, 0)),
         pl.BlockSpec((1, H), lambda i: (0, 0))],
        [pl.BlockSpec((NB, H), lambda i: (i, 0)),
         pl.BlockSpec((NB, H), lambda i: (i, 0))],
        [jax.ShapeDtypeStruct((NP, H), jnp.float32),
         jax.ShapeDtypeStruct((NP, H), jnp.float32)],
        (accp2[0], accp2[1], sp2[0][:, None], sp2[1][:, None], asad2, h2,
         b2[None, :], fc1_W[:H], fc1_W[H:], fc1_b[None, :]),
    )
    z = _sc_mlp(srcp, dstp, ps, pd)
    return _tc_out(z, fc2_W, fc2_b[None, :])


# speedup vs baseline: 40.7768x; 1.3086x over previous
import functools

import jax
import jax.numpy as jnp
from jax import lax
from jax.experimental import pallas as pl
from jax.experimental.pallas import tpu as pltpu
from jax.experimental.pallas import tpu_sc as plsc

N = 10000
E = 320000
D = 128
H = 16
NC = 16

NP = 10240
NB = 1024
NPB = NP // 16

NTILES = 32
ROWS = 81
CH = 128
PT = ROWS * CH
EP = NTILES * PT
EB = 2000
EPR = EP // 8
ER = E * NC // 128
PTR = PT // 8

_MESH = plsc.VectorSubcoreMesh(
    core_axis_name="c", subcore_axis_name="s", num_cores=2, num_subcores=16
)
_SC_PARAMS = pltpu.CompilerParams(
    needs_layout_passes=False, use_tc_tiling_on_sc=False
)


def _tc_front_body(x_ref, w_ref, a_ref, h_ref, asad_ref):
    h = jnp.dot(x_ref[...], w_ref[...], preferred_element_type=jnp.float32)
    h_ref[...] = h
    asad_ref[...] = jnp.dot(h, a_ref[...], preferred_element_type=jnp.float32)


def _tc_front(xp, W1, A1):
    return pl.pallas_call(
        _tc_front_body,
        grid=(NP // NB,),
        in_specs=[
            pl.BlockSpec((NB, D), lambda i: (i, 0)),
            pl.BlockSpec((D, H), lambda i: (0, 0)),
            pl.BlockSpec((H, 2), lambda i: (0, 0)),
        ],
        out_specs=[
            pl.BlockSpec((NB, H), lambda i: (i, 0)),
            pl.BlockSpec((NB, 2), lambda i: (i, 0)),
        ],
        out_shape=[
            jax.ShapeDtypeStruct((NP, H), jnp.float32),
            jax.ShapeDtypeStruct((NP, 2), jnp.float32),
        ],
    )(xp, W1, A1)


def _combine(acc0_ref, acc1_ref, s0_ref, s1_ref, asad_ref, h_ref, b_ref):
    al = asad_ref[...]
    lg = al[:, 0:1] + al[:, 1:2]
    lg = jnp.where(lg >= 0.0, lg, 0.2 * lg)
    ex = jnp.exp(lg)
    acc = acc0_ref[...] + acc1_ref[...] + ex * h_ref[...]
    st = s0_ref[...] + s1_ref[...] + ex
    g = acc / (st + 1e-16) + b_ref[...]
    return jnp.where(g > 0.0, g, jnp.exp(g) - 1.0)


def _tc_mid_body(acc0_ref, acc1_ref, s0_ref, s1_ref, asad_ref, h_ref, b_ref,
                 w_ref, a_ref, h2_ref, asad2_ref):
    hin = _combine(acc0_ref, acc1_ref, s0_ref, s1_ref, asad_ref, h_ref, b_ref)
    h2 = jnp.dot(hin, w_ref[...], preferred_element_type=jnp.float32)
    h2_ref[...] = h2
    asad2_ref[...] = jnp.dot(h2, a_ref[...], preferred_element_type=jnp.float32)


def _tc_tail_body(acc0_ref, acc1_ref, s0_ref, s1_ref, asad_ref, h_ref, b_ref,
                  w1a_ref, w1b_ref, fb_ref, ps_ref, pd_ref):
    hf = _combine(acc0_ref, acc1_ref, s0_ref, s1_ref, asad_ref, h_ref, b_ref)
    ps_ref[...] = jnp.dot(hf, w1a_ref[...], preferred_element_type=jnp.float32)
    pd_ref[...] = (
        jnp.dot(hf, w1b_ref[...], preferred_element_type=jnp.float32)
        + fb_ref[...]
    )


def _tc_combine_call(body, extra_specs, out_specs, out_shape, args):
    return pl.pallas_call(
        body,
        grid=(NP // NB,),
        in_specs=[
            pl.BlockSpec((NB, H), lambda i: (i, 0)),
            pl.BlockSpec((NB, H), lambda i: (i, 0)),
            pl.BlockSpec((NB, 1), lambda i: (i, 0)),
            pl.BlockSpec((NB, 1), lambda i: (i, 0)),
            pl.BlockSpec((NB, 2), lambda i: (i, 0)),
            pl.BlockSpec((NB, H), lambda i: (i, 0)),
            pl.BlockSpec((1, H), lambda i: (0, 0)),
        ] + extra_specs,
        out_specs=out_specs,
        out_shape=out_shape,
    )(*args)


def _tc_out_body(z_ref, bd_ref, bd1_ref, b_ref, o_ref):
    y = jnp.dot(z_ref[...], bd_ref[...], preferred_element_type=jnp.float32)
    y = y + b_ref[...]
    ey = jnp.exp(y)
    ssum = jnp.dot(ey, bd1_ref[...], preferred_element_type=jnp.float32)
    o_ref[...] = y - jnp.log(ssum)


def _tc_out(z2, bd, bd1, btile):
    return pl.pallas_call(
        _tc_out_body,
        grid=(ER // EB,),
        in_specs=[
            pl.BlockSpec((EB, 128), lambda i: (i, 0)),
            pl.BlockSpec((128, 128), lambda i: (0, 0)),
            pl.BlockSpec((128, 128), lambda i: (0, 0)),
            pl.BlockSpec((1, 128), lambda i: (0, 0)),
        ],
        out_specs=pl.BlockSpec((EB, 128), lambda i: (i, 0)),
        out_shape=jax.ShapeDtypeStruct((ER, 128), jnp.float32),
    )(z2, bd, bd1, btile)


def _sc_agg_body(srcr, dstr, a_s, a_d, htab, accp, sp,
                 src_v, dst_v, as_v, ad_v,
                 hb0, hb1, hb2, exb0, exb1, exb2,
                 zrow, zcol, acc_sh, s_sh,
                 gs0, gs1, gs2, ss0, ss1, ss2):
    c = lax.axis_index("c")
    s = lax.axis_index("s")
    t = c * 16 + s
    zv = jnp.zeros((16,), jnp.float32)
    hbufs = (hb0, hb1, hb2)
    exbufs = (exb0, exb1, exb2)
    gsems = (gs0, gs1, gs2)
    ssems = (ss0, ss1, ss2)

    def zfill(i, carry):
        zrow[i] = zv
        return carry

    lax.fori_loop(0, NPB, zfill, 0)

    def zfill1(i, carry):
        zcol[pl.ds(i * 16, 16)] = zv
        return carry

    lax.fori_loop(0, NPB // 16, zfill1, 0)
    pltpu.sync_copy(zrow, acc_sh.at[pl.ds(s * NPB, NPB)])
    pltpu.sync_copy(zcol, s_sh.at[pl.ds(s * NPB, NPB)])
    pltpu.sync_copy(a_s, as_v)
    pltpu.sync_copy(a_d, ad_v)
    pltpu.sync_copy(srcr.at[t], src_v)
    pltpu.sync_copy(dstr.at[t], dst_v)
    plsc.subcore_barrier()

    base = t * PT

    def gissue(j, b):
        pltpu.async_copy(htab.at[src_v.at[j]], hbufs[b], gsems[b])

    def swait(b):
        pltpu.make_async_copy(hbufs[b], acc_sh.at[dst_v.at[0]], ssems[b]).wait()
        pltpu.make_async_copy(exbufs[b], s_sh.at[dst_v.at[0]], ssems[b]).wait()

    def section(j, b):
        hb = hbufs[b]
        exb = exbufs[b]
        pltpu.make_async_copy(htab.at[src_v.at[j]], hb, gsems[b]).wait()
        for v in range(CH // 16):
            sl = pl.ds(v * 16, 16)
            sidx = src_v[j, sl]
            didx = dst_v[j, sl]
            av = plsc.load_gather(as_v, [sidx])
            bv = plsc.load_gather(ad_v, [didx])
            e = av + bv
            e = jnp.where(e >= 0.0, e, 0.2 * e)
            ex = jnp.exp(e)
            gid = base + j * CH + v * 16 + lax.broadcasted_iota(jnp.int32, (16,), 0)
            ex = jnp.where(gid < E, ex, 0.0)
            exb[sl] = ex
            for l in range(16):
                k = v * 16 + l
                hb[k] = hb[k] * ex[l]
        pltpu.async_copy(hb, acc_sh.at[dst_v.at[j]], ssems[b], add=True)
        pltpu.async_copy(exb, s_sh.at[dst_v.at[j]], ssems[b], add=True)
        nb = (b + 2) % 3

        @pl.when(j < ROWS - 2)
        def _():
            @pl.when(j >= 1)
            def _():
                swait(nb)

            gissue(j + 2, nb)

    gissue(0, 0)
    gissue(1, 1)

    def gbody(g, carry):
        for k in range(3):
            section(3 * g + k, k)
        return carry

    lax.fori_loop(0, ROWS // 3, gbody, 0)
    swait(0)
    swait(1)
    swait(2)
    plsc.subcore_barrier()
    pltpu.sync_copy(acc_sh.at[pl.ds(s * NPB, NPB)], accp.at[c, pl.ds(s * NPB, NPB)])
    pltpu.sync_copy(s_sh.at[pl.ds(s * NPB, NPB)], sp.at[c, pl.ds(s * NPB, NPB)])


_sc_agg = functools.partial(
    pl.kernel,
    out_type=[
        jax.ShapeDtypeStruct((2, NP, H), jnp.float32),
        jax.ShapeDtypeStruct((2, NP), jnp.float32),
    ],
    mesh=_MESH,
    compiler_params=_SC_PARAMS,
    scratch_types=[
        pltpu.VMEM((ROWS, CH), jnp.int32),
        pltpu.VMEM((ROWS, CH), jnp.int32),
        pltpu.VMEM((NP,), jnp.float32),
        pltpu.VMEM((NP,), jnp.float32),
        pltpu.VMEM((CH, H), jnp.float32),
        pltpu.VMEM((CH, H), jnp.float32),
        pltpu.VMEM((CH, H), jnp.float32),
        pltpu.VMEM((CH,), jnp.float32),
        pltpu.VMEM((CH,), jnp.float32),
        pltpu.VMEM((CH,), jnp.float32),
        pltpu.VMEM((NPB, H), jnp.float32),
        pltpu.VMEM((NPB,), jnp.float32),
        pltpu.VMEM_SHARED((NP, H), jnp.float32),
        pltpu.VMEM_SHARED((NP,), jnp.float32),
        pltpu.SemaphoreType.DMA,
        pltpu.SemaphoreType.DMA,
        pltpu.SemaphoreType.DMA,
        pltpu.SemaphoreType.DMA,
        pltpu.SemaphoreType.DMA,
        pltpu.SemaphoreType.DMA,
    ],
)(_sc_agg_body)


def _sc_mlp_body(srcr, dstr, ps_tab, pd_tab, z_out,
                 src_v, dst_v, pa0, pa1, pa2, pb0, pb1, pb2,
                 wb0, wb1, wb2, gs0, gs1, gs2, ws0, ws1, ws2):
    c = lax.axis_index("c")
    s = lax.axis_index("s")
    t = c * 16 + s
    pas = (pa0, pa1, pa2)
    pbs = (pb0, pb1, pb2)
    wbs = (wb0, wb1, wb2)
    gsems = (gs0, gs1, gs2)
    wsems = (ws0, ws1, ws2)
    pltpu.sync_copy(srcr.at[t], src_v)
    pltpu.sync_copy(dstr.at[t], dst_v)

    def gissue(j, b):
        pltpu.async_copy(ps_tab.at[src_v.at[j]], pas[b], gsems[b])
        pltpu.async_copy(pd_tab.at[dst_v.at[j]], pbs[b], gsems[b])

    def wwait(b):
        pltpu.make_async_copy(wbs[b], z_out.at[pl.ds(t * PTR, CH // 8)], wsems[b]).wait()

    def section(j, b):
        pa = pas[b]
        pb = pbs[b]
        wb = wbs[b]
        pltpu.make_async_copy(ps_tab.at[src_v.at[j]], pa, gsems[b]).wait()
        pltpu.make_async_copy(pd_tab.at[dst_v.at[j]], pb, gsems[b]).wait()
        for k in range(CH):
            wb[k // 8, pl.ds((k % 8) * H, H)] = jnp.maximum(pa[k] + pb[k], 0.0)
        pltpu.async_copy(wb, z_out.at[pl.ds(t * PTR + j * (CH // 8), CH // 8)], wsems[b])
        nb = (b + 2) % 3

        @pl.when(j < ROWS - 2)
        def _():
            @pl.when(j >= 1)
            def _():
                wwait(nb)

            gissue(j + 2, nb)

    gissue(0, 0)
    gissue(1, 1)

    def gbody(g, carry):
        for k in range(3):
            section(3 * g + k, k)
        return carry

    lax.fori_loop(0, ROWS // 3, gbody, 0)
    wwait(0)
    wwait(1)
    wwait(2)


_sc_mlp = functools.partial(
    pl.kernel,
    out_type=jax.ShapeDtypeStruct((EPR, 128), jnp.float32),
    mesh=_MESH,
    compiler_params=_SC_PARAMS,
    scratch_types=[
        pltpu.VMEM((ROWS, CH), jnp.int32),
        pltpu.VMEM((ROWS, CH), jnp.int32),
        pltpu.VMEM((CH, H), jnp.float32),
        pltpu.VMEM((CH, H), jnp.float32),
        pltpu.VMEM((CH, H), jnp.float32),
        pltpu.VMEM((CH, H), jnp.float32),
        pltpu.VMEM((CH, H), jnp.float32),
        pltpu.VMEM((CH, H), jnp.float32),
        pltpu.VMEM((CH // 8, 128), jnp.float32),
        pltpu.VMEM((CH // 8, 128), jnp.float32),
        pltpu.VMEM((CH // 8, 128), jnp.float32),
        pltpu.SemaphoreType.DMA,
        pltpu.SemaphoreType.DMA,
        pltpu.SemaphoreType.DMA,
        pltpu.SemaphoreType.DMA,
        pltpu.SemaphoreType.DMA,
        pltpu.SemaphoreType.DMA,
    ],
)(_sc_mlp_body)


def kernel(x, edge_index, W1, a_src1, a_dst1, b1, W2, a_src2, a_dst2, b2,
           fc1_W, fc1_b, fc2_W, fc2_b):
    src = edge_index[0].astype(jnp.int32)
    dst = edge_index[1].astype(jnp.int32)
    srcp = jnp.pad(src, (0, EP - E)).reshape(NTILES, ROWS, CH)
    dstp = jnp.pad(dst, (0, EP - E)).reshape(NTILES, ROWS, CH)
    xp = jnp.pad(x, ((0, NP - N), (0, 0)))
    A1 = jnp.stack([a_src1, a_dst1], axis=1)
    A2 = jnp.stack([a_src2, a_dst2], axis=1)

    h1, asad1 = _tc_front(xp, W1, A1)
    accp1, sp1 = _sc_agg(srcp, dstp, asad1[:, 0], asad1[:, 1], h1)
    h2, asad2 = _tc_combine_call(
        _tc_mid_body,
        [pl.BlockSpec((H, H), lambda i: (0, 0)),
         pl.BlockSpec((H, 2), lambda i: (0, 0))],
        [pl.BlockSpec((NB, H), lambda i: (i, 0)),
         pl.BlockSpec((NB, 2), lambda i: (i, 0))],
        [jax.ShapeDtypeStruct((NP, H), jnp.float32),
         jax.ShapeDtypeStruct((NP, 2), jnp.float32)],
        (accp1[0], accp1[1], sp1[0][:, None], sp1[1][:, None], asad1, h1,
         b1[None, :], W2, A2),
    )
    accp2, sp2 = _sc_agg(srcp, dstp, asad2[:, 0], asad2[:, 1], h2)
    ps, pd = _tc_combine_call(
        _tc_tail_body,
        [pl.BlockSpec((H, H), lambda i: (0, 0)),
         pl.BlockSpec((H, H), lambda i: (0, 0)),
         pl.BlockSpec((1, H), lambda i: (0, 0))],
        [pl.BlockSpec((NB, H), lambda i: (i, 0)),
         pl.BlockSpec((NB, H), lambda i: (i, 0))],
        [jax.ShapeDtypeStruct((NP, H), jnp.float32),
         jax.ShapeDtypeStruct((NP, H), jnp.float32)],
        (accp2[0], accp2[1], sp2[0][:, None], sp2[1][:, None], asad2, h2,
         b2[None, :], fc1_W[:H], fc1_W[H:], fc1_b[None, :]),
    )
    z2 = _sc_mlp(srcp, dstp, ps, pd)
    eye8 = jnp.eye(8, dtype=jnp.float32)
    bd = jnp.kron(eye8, fc2_W)
    bd1 = jnp.kron(eye8, jnp.ones((H, NC), jnp.float32))
    btile = jnp.tile(fc2_b, 8)[None, :]
    out2 = _tc_out(z2, bd, bd1, btile)
    return out2.reshape(E, NC)


# speedup vs baseline: 54.5450x; 1.3376x over previous
import functools

import jax
import jax.numpy as jnp
from jax import lax
from jax.experimental import pallas as pl
from jax.experimental.pallas import tpu as pltpu
from jax.experimental.pallas import tpu_sc as plsc

N = 10000
E = 320000
D = 128
H = 16
NC = 16

NP = 10240
NB = 1024
NPB = NP // 16

NTILES = 32
CH = 128
ERWS = E // CH
BR = ERWS // NTILES
XT = ERWS % NTILES
GR = BR // 3
EB = 2000
ER = E * NC // 128

_MESH = plsc.VectorSubcoreMesh(
    core_axis_name="c", subcore_axis_name="s", num_cores=2, num_subcores=16
)
_SC_PARAMS = pltpu.CompilerParams(
    needs_layout_passes=False, use_tc_tiling_on_sc=False
)


def _tc_front_body(x_ref, a_ref, h_ref, as_ref, ad_ref):
    h = jnp.dot(x_ref[...], a_ref[..., :D].T, preferred_element_type=jnp.float32)
    h_ref[...] = h
    asad = jnp.dot(h, a_ref[..., D:D + 2], preferred_element_type=jnp.float32)
    as_ref[...] = asad[:, 0]
    ad_ref[...] = asad[:, 1]


def _tc_front(x, wa):
    return pl.pallas_call(
        _tc_front_body,
        grid=(NP // NB,),
        in_specs=[
            pl.BlockSpec((NB, D), lambda i: (i, 0)),
            pl.BlockSpec((H, D + 2), lambda i: (0, 0)),
        ],
        out_specs=[
            pl.BlockSpec((NB, H), lambda i: (i, 0)),
            pl.BlockSpec((NB,), lambda i: (i,)),
            pl.BlockSpec((NB,), lambda i: (i,)),
        ],
        out_shape=[
            jax.ShapeDtypeStruct((NP, H), jnp.float32),
            jax.ShapeDtypeStruct((NP,), jnp.float32),
            jax.ShapeDtypeStruct((NP,), jnp.float32),
        ],
    )(x, wa)


def _combine(acc0_ref, acc1_ref, s0_ref, s1_ref, as_ref, ad_ref, h_ref, b_ref):
    lg = as_ref[...] + ad_ref[...]
    lg = jnp.where(lg >= 0.0, lg, 0.2 * lg)
    ex = jnp.exp(lg)
    acc = acc0_ref[...] + acc1_ref[...] + ex[:, None] * h_ref[...]
    st = s0_ref[...] + s1_ref[...] + ex
    g = acc / (st[:, None] + 1e-16) + b_ref[...]
    return jnp.where(g > 0.0, g, jnp.exp(g) - 1.0)


def _tc_mid_body(acc0_ref, acc1_ref, s0_ref, s1_ref, as_ref, ad_ref, h_ref,
                 b_ref, w_ref, a_ref, h2_ref, as2_ref, ad2_ref):
    hin = _combine(acc0_ref, acc1_ref, s0_ref, s1_ref, as_ref, ad_ref, h_ref,
                   b_ref)
    h2 = jnp.dot(hin, w_ref[...], preferred_element_type=jnp.float32)
    h2_ref[...] = h2
    asad2 = jnp.dot(h2, a_ref[...], preferred_element_type=jnp.float32)
    as2_ref[...] = asad2[:, 0]
    ad2_ref[...] = asad2[:, 1]


def _tc_tail_body(acc0_ref, acc1_ref, s0_ref, s1_ref, as_ref, ad_ref, h_ref,
                  b_ref, w1a_ref, w1b_ref, fb_ref, ps_ref, pd_ref):
    hf = _combine(acc0_ref, acc1_ref, s0_ref, s1_ref, as_ref, ad_ref, h_ref,
                  b_ref)
    ps_ref[...] = jnp.dot(hf, w1a_ref[...], preferred_element_type=jnp.float32)
    pd_ref[...] = (
        jnp.dot(hf, w1b_ref[...], preferred_element_type=jnp.float32)
        + fb_ref[...]
    )


def _tc_combine_call(body, extra_specs, out_specs, out_shape, args):
    return pl.pallas_call(
        body,
        grid=(NP // NB,),
        in_specs=[
            pl.BlockSpec((NB, H), lambda i: (i, 0)),
            pl.BlockSpec((NB, H), lambda i: (i, 0)),
            pl.BlockSpec((NB,), lambda i: (i,)),
            pl.BlockSpec((NB,), lambda i: (i,)),
            pl.BlockSpec((NB,), lambda i: (i,)),
            pl.BlockSpec((NB,), lambda i: (i,)),
            pl.BlockSpec((NB, H), lambda i: (i, 0)),
            pl.BlockSpec((1, H), lambda i: (0, 0)),
        ] + extra_specs,
        out_specs=out_specs,
        out_shape=out_shape,
    )(*args)


def _tc_out_body(z_ref, bd_ref, bd1_ref, b_ref, o_ref):
    y = jnp.dot(z_ref[...], bd_ref[...], preferred_element_type=jnp.float32)
    y = y + b_ref[...]
    ey = jnp.exp(y)
    ssum = jnp.dot(ey, bd1_ref[...], preferred_element_type=jnp.float32)
    o_ref[...] = y - jnp.log(ssum)


def _tc_out(z2, bd, bd1, btile):
    return pl.pallas_call(
        _tc_out_body,
        grid=(ER // EB,),
        in_specs=[
            pl.BlockSpec((EB, 128), lambda i: (i, 0)),
            pl.BlockSpec((128, 128), lambda i: (0, 0)),
            pl.BlockSpec((128, 128), lambda i: (0, 0)),
            pl.BlockSpec((1, 128), lambda i: (0, 0)),
        ],
        out_specs=pl.BlockSpec((EB, 128), lambda i: (i, 0)),
        out_shape=jax.ShapeDtypeStruct((ER, 128), jnp.float32),
    )(z2, bd, bd1, btile)


def _sc_agg_body(srcdst, a_s, a_d, htab, acc0, acc1, s0, s1,
                 src_v, dst_v, as_v, ad_v,
                 hb0, hb1, hb2, exb0, exb1, exb2,
                 zrow, zcol, acc_sh, s_sh,
                 gs0, gs1, gs2, ss0, ss1, ss2):
    c = lax.axis_index("c")
    s = lax.axis_index("s")
    t = c * 16 + s
    nr = BR + (t < XT).astype(jnp.int32)
    roff = BR * t + jnp.minimum(t, XT)
    zv = jnp.zeros((16,), jnp.float32)
    hbufs = (hb0, hb1, hb2)
    exbufs = (exb0, exb1, exb2)
    gsems = (gs0, gs1, gs2)
    ssems = (ss0, ss1, ss2)

    def zfill(i, carry):
        zrow[i] = zv
        return carry

    lax.fori_loop(0, NPB, zfill, 0)

    def zfill1(i, carry):
        zcol[pl.ds(i * 16, 16)] = zv
        return carry

    lax.fori_loop(0, NPB // 16, zfill1, 0)
    pltpu.sync_copy(zrow, acc_sh.at[pl.ds(s * NPB, NPB)])
    pltpu.sync_copy(zcol, s_sh.at[pl.ds(s * NPB, NPB)])
    pltpu.sync_copy(a_s, as_v)
    pltpu.sync_copy(a_d, ad_v)
    pltpu.sync_copy(srcdst.at[pl.ds(roff, BR)], src_v.at[pl.ds(0, BR)])
    pltpu.sync_copy(srcdst.at[pl.ds(ERWS + roff, BR)], dst_v.at[pl.ds(0, BR)])

    @pl.when(t < XT)
    def _():
        pltpu.sync_copy(srcdst.at[pl.ds(roff + BR, 1)], src_v.at[pl.ds(BR, 1)])
        pltpu.sync_copy(srcdst.at[pl.ds(ERWS + roff + BR, 1)],
                        dst_v.at[pl.ds(BR, 1)])

    plsc.subcore_barrier()

    def gissue(j, b):
        pltpu.async_copy(htab.at[src_v.at[j]], hbufs[b], gsems[b])

    def swait(b):
        pltpu.make_async_copy(hbufs[b], acc_sh.at[dst_v.at[0]], ssems[b]).wait()
        pltpu.make_async_copy(exbufs[b], s_sh.at[dst_v.at[0]], ssems[b]).wait()

    def section(j, b):
        hb = hbufs[b]
        exb = exbufs[b]
        pltpu.make_async_copy(htab.at[src_v.at[j]], hb, gsems[b]).wait()
        for v in range(CH // 16):
            sl = pl.ds(v * 16, 16)
            sidx = src_v[j, sl]
            didx = dst_v[j, sl]
            av = plsc.load_gather(as_v, [sidx])
            bv = plsc.load_gather(ad_v, [didx])
            e = av + bv
            e = jnp.where(e >= 0.0, e, 0.2 * e)
            ex = jnp.exp(e)
            exb[sl] = ex
            for l in range(16):
                k = v * 16 + l
                hb[k] = hb[k] * ex[l]
        pltpu.async_copy(hb, acc_sh.at[dst_v.at[j]], ssems[b], add=True)
        pltpu.async_copy(exb, s_sh.at[dst_v.at[j]], ssems[b], add=True)
        nb = (b + 2) % 3

        @pl.when(j < nr - 2)
        def _():
            @pl.when(j >= 1)
            def _():
                swait(nb)

            gissue(j + 2, nb)

    gissue(0, 0)
    gissue(1, 1)

    def gbody(g, carry):
        for k in range(3):
            section(3 * g + k, k)
        return carry

    lax.fori_loop(0, GR, gbody, 0)

    @pl.when(nr > BR)
    def _():
        section(BR, BR % 3)

    swait(0)
    swait(1)
    swait(2)
    plsc.subcore_barrier()

    @pl.when(c == 0)
    def _():
        pltpu.sync_copy(acc_sh.at[pl.ds(s * NPB, NPB)], acc0.at[pl.ds(s * NPB, NPB)])
        pltpu.sync_copy(s_sh.at[pl.ds(s * NPB, NPB)], s0.at[pl.ds(s * NPB, NPB)])

    @pl.when(c == 1)
    def _():
        pltpu.sync_copy(acc_sh.at[pl.ds(s * NPB, NPB)], acc1.at[pl.ds(s * NPB, NPB)])
        pltpu.sync_copy(s_sh.at[pl.ds(s * NPB, NPB)], s1.at[pl.ds(s * NPB, NPB)])


_sc_agg = functools.partial(
    pl.kernel,
    out_type=[
        jax.ShapeDtypeStruct((NP, H), jnp.float32),
        jax.ShapeDtypeStruct((NP, H), jnp.float32),
        jax.ShapeDtypeStruct((NP,), jnp.float32),
        jax.ShapeDtypeStruct((NP,), jnp.float32),
    ],
    mesh=_MESH,
    compiler_params=_SC_PARAMS,
    scratch_types=[
        pltpu.VMEM((BR + 1, CH), jnp.int32),
        pltpu.VMEM((BR + 1, CH), jnp.int32),
        pltpu.VMEM((NP,), jnp.float32),
        pltpu.VMEM((NP,), jnp.float32),
        pltpu.VMEM((CH, H), jnp.float32),
        pltpu.VMEM((CH, H), jnp.float32),
        pltpu.VMEM((CH, H), jnp.float32),
        pltpu.VMEM((CH,), jnp.float32),
        pltpu.VMEM((CH,), jnp.float32),
        pltpu.VMEM((CH,), jnp.float32),
        pltpu.VMEM((NPB, H), jnp.float32),
        pltpu.VMEM((NPB,), jnp.float32),
        pltpu.VMEM_SHARED((NP, H), jnp.float32),
        pltpu.VMEM_SHARED((NP,), jnp.float32),
        pltpu.SemaphoreType.DMA,
        pltpu.SemaphoreType.DMA,
        pltpu.SemaphoreType.DMA,
        pltpu.SemaphoreType.DMA,
        pltpu.SemaphoreType.DMA,
        pltpu.SemaphoreType.DMA,
    ],
)(_sc_agg_body)


def _sc_mlp_body(srcdst, ps_tab, pd_tab, z_out,
                 src_v, dst_v, pa0, pa1, pa2, pb0, pb1, pb2,
                 wb0, wb1, wb2, gs0, gs1, gs2, ws0, ws1, ws2):
    c = lax.axis_index("c")
    s = lax.axis_index("s")
    t = c * 16 + s
    nr = BR + (t < XT).astype(jnp.int32)
    roff = BR * t + jnp.minimum(t, XT)
    zbase = roff * (CH // 8)
    pas = (pa0, pa1, pa2)
    pbs = (pb0, pb1, pb2)
    wbs = (wb0, wb1, wb2)
    gsems = (gs0, gs1, gs2)
    wsems = (ws0, ws1, ws2)
    pltpu.sync_copy(srcdst.at[pl.ds(roff, BR)], src_v.at[pl.ds(0, BR)])
    pltpu.sync_copy(srcdst.at[pl.ds(ERWS + roff, BR)], dst_v.at[pl.ds(0, BR)])

    @pl.when(t < XT)
    def _():
        pltpu.sync_copy(srcdst.at[pl.ds(roff + BR, 1)], src_v.at[pl.ds(BR, 1)])
        pltpu.sync_copy(srcdst.at[pl.ds(ERWS + roff + BR, 1)],
                        dst_v.at[pl.ds(BR, 1)])

    def gissue(j, b):
        pltpu.async_copy(ps_tab.at[src_v.at[j]], pas[b], gsems[b])
        pltpu.async_copy(pd_tab.at[dst_v.at[j]], pbs[b], gsems[b])

    def wwait(b):
        pltpu.make_async_copy(wbs[b], z_out.at[pl.ds(0, CH // 8)], wsems[b]).wait()

    def section(j, b):
        pa = pas[b]
        pb = pbs[b]
        wb = wbs[b]
        pltpu.make_async_copy(ps_tab.at[src_v.at[j]], pa, gsems[b]).wait()
        pltpu.make_async_copy(pd_tab.at[dst_v.at[j]], pb, gsems[b]).wait()
        for k in range(CH):
            wb[k // 8, pl.ds((k % 8) * H, H)] = jnp.maximum(pa[k] + pb[k], 0.0)
        pltpu.async_copy(wb, z_out.at[pl.ds(zbase + j * (CH // 8), CH // 8)], wsems[b])
        nb = (b + 2) % 3

        @pl.when(j < nr - 2)
        def _():
            @pl.when(j >= 1)
            def _():
                wwait(nb)

            gissue(j + 2, nb)

    gissue(0, 0)
    gissue(1, 1)

    def gbody(g, carry):
        for k in range(3):
            section(3 * g + k, k)
        return carry

    lax.fori_loop(0, GR, gbody, 0)

    @pl.when(nr > BR)
    def _():
        section(BR, BR % 3)

    wwait(0)
    wwait(1)
    wwait(2)


_sc_mlp = functools.partial(
    pl.kernel,
    out_type=jax.ShapeDtypeStruct((ER, 128), jnp.float32),
    mesh=_MESH,
    compiler_params=_SC_PARAMS,
    scratch_types=[
        pltpu.VMEM((BR + 1, CH), jnp.int32),
        pltpu.VMEM((BR + 1, CH), jnp.int32),
        pltpu.VMEM((CH, H), jnp.float32),
        pltpu.VMEM((CH, H), jnp.float32),
        pltpu.VMEM((CH, H), jnp.float32),
        pltpu.VMEM((CH, H), jnp.float32),
        pltpu.VMEM((CH, H), jnp.float32),
        pltpu.VMEM((CH, H), jnp.float32),
        pltpu.VMEM((CH // 8, 128), jnp.float32),
        pltpu.VMEM((CH // 8, 128), jnp.float32),
        pltpu.VMEM((CH // 8, 128), jnp.float32),
        pltpu.SemaphoreType.DMA,
        pltpu.SemaphoreType.DMA,
        pltpu.SemaphoreType.DMA,
        pltpu.SemaphoreType.DMA,
        pltpu.SemaphoreType.DMA,
        pltpu.SemaphoreType.DMA,
    ],
)(_sc_mlp_body)


def kernel(x, edge_index, W1, a_src1, a_dst1, b1, W2, a_src2, a_dst2, b2,
           fc1_W, fc1_b, fc2_W, fc2_b):
    srcdst = edge_index.astype(jnp.int32).reshape(2 * ERWS, CH)
    wa1 = jnp.concatenate(
        [W1.T, a_src1[:, None], a_dst1[:, None]], axis=1)
    A2 = jnp.stack([a_src2, a_dst2], axis=1)

    h1, as1, ad1 = _tc_front(x, wa1)
    acc0, acc1, s0, s1 = _sc_agg(srcdst, as1, ad1, h1)
    h2, as2, ad2 = _tc_combine_call(
        _tc_mid_body,
        [pl.BlockSpec((H, H), lambda i: (0, 0)),
         pl.BlockSpec((H, 2), lambda i: (0, 0))],
        [pl.BlockSpec((NB, H), lambda i: (i, 0)),
         pl.BlockSpec((NB,), lambda i: (i,)),
         pl.BlockSpec((NB,), lambda i: (i,))],
        [jax.ShapeDtypeStruct((NP, H), jnp.float32),
         jax.ShapeDtypeStruct((NP,), jnp.float32),
         jax.ShapeDtypeStruct((NP,), jnp.float32)],
        (acc0, acc1, s0, s1, as1, ad1, h1, b1[None, :], W2, A2),
    )
    acc0b, acc1b, s0b, s1b = _sc_agg(srcdst, as2, ad2, h2)
    ps, pd = _tc_combine_call(
        _tc_tail_body,
        [pl.BlockSpec((H, H), lambda i: (0, 0)),
         pl.BlockSpec((H, H), lambda i: (0, 0)),
         pl.BlockSpec((1, H), lambda i: (0, 0))],
        [pl.BlockSpec((NB, H), lambda i: (i, 0)),
         pl.BlockSpec((NB, H), lambda i: (i, 0))],
        [jax.ShapeDtypeStruct((NP, H), jnp.float32),
         jax.ShapeDtypeStruct((NP, H), jnp.float32)],
        (acc0b, acc1b, s0b, s1b, as2, ad2, h2, b2[None, :],
         fc1_W[:H], fc1_W[H:], fc1_b[None, :]),
    )
    z2 = _sc_mlp(srcdst, ps, pd)
    eye8 = jnp.eye(8, dtype=jnp.float32)
    bd = jnp.kron(eye8, fc2_W)
    bd1 = jnp.kron(eye8, jnp.ones((H, NC), jnp.float32))
    btile = jnp.tile(fc2_b, 8)[None, :]
    out2 = _tc_out(z2, bd, bd1, btile)
    return out2.reshape(E, NC)


# speedup vs baseline: 54.5591x; 1.0003x over previous
import functools

import jax
import jax.numpy as jnp
from jax import lax
from jax.experimental import pallas as pl
from jax.experimental.pallas import tpu as pltpu
from jax.experimental.pallas import tpu_sc as plsc

N = 10000
E = 320000
D = 128
H = 16
NC = 16

NP = 10240
NB = 1024
NPB = NP // 16

NTILES = 32
CH = 128
ERWS = E // CH
BR = ERWS // NTILES
XT = ERWS % NTILES
GR = BR // 3
EB = 2000
ER = E * NC // 128

_MESH = plsc.VectorSubcoreMesh(
    core_axis_name="c", subcore_axis_name="s", num_cores=2, num_subcores=16
)
_SC_PARAMS = pltpu.CompilerParams(
    needs_layout_passes=False, use_tc_tiling_on_sc=False
)


def _tc_front_body(x_ref, a_ref, h_ref, as_ref, ad_ref):
    h = jnp.dot(x_ref[...], a_ref[..., :D].T, preferred_element_type=jnp.float32)
    h_ref[...] = h
    asad = jnp.dot(h, a_ref[..., D:D + 2], preferred_element_type=jnp.float32)
    as_ref[...] = asad[:, 0]
    ad_ref[...] = asad[:, 1]


def _tc_front(x, wa):
    return pl.pallas_call(
        _tc_front_body,
        grid=(NP // NB,),
        in_specs=[
            pl.BlockSpec((NB, D), lambda i: (i, 0)),
            pl.BlockSpec((H, D + 2), lambda i: (0, 0)),
        ],
        out_specs=[
            pl.BlockSpec((NB, H), lambda i: (i, 0)),
            pl.BlockSpec((NB,), lambda i: (i,)),
            pl.BlockSpec((NB,), lambda i: (i,)),
        ],
        out_shape=[
            jax.ShapeDtypeStruct((NP, H), jnp.float32),
            jax.ShapeDtypeStruct((NP,), jnp.float32),
            jax.ShapeDtypeStruct((NP,), jnp.float32),
        ],
    )(x, wa)


def _combine(acc0_ref, acc1_ref, s0_ref, s1_ref, as_ref, ad_ref, h_ref, b_ref):
    lg = as_ref[...] + ad_ref[...]
    lg = jnp.where(lg >= 0.0, lg, 0.2 * lg)
    ex = jnp.exp(lg)
    acc = acc0_ref[...] + acc1_ref[...] + ex[:, None] * h_ref[...]
    st = s0_ref[...] + s1_ref[...] + ex
    g = acc / (st[:, None] + 1e-16) + b_ref[...]
    return jnp.where(g > 0.0, g, jnp.exp(g) - 1.0)


def _tc_mid_body(acc0_ref, acc1_ref, s0_ref, s1_ref, as_ref, ad_ref, h_ref,
                 b_ref, w_ref, a_ref, h2_ref, as2_ref, ad2_ref):
    hin = _combine(acc0_ref, acc1_ref, s0_ref, s1_ref, as_ref, ad_ref, h_ref,
                   b_ref)
    h2 = jnp.dot(hin, w_ref[...], preferred_element_type=jnp.float32)
    h2_ref[...] = h2
    asad2 = jnp.dot(h2, a_ref[...], preferred_element_type=jnp.float32)
    as2_ref[...] = asad2[:, 0]
    ad2_ref[...] = asad2[:, 1]


def _tc_tail_body(acc0_ref, acc1_ref, s0_ref, s1_ref, as_ref, ad_ref, h_ref,
                  b_ref, w1a_ref, w1b_ref, fb_ref, ps_ref, pd_ref):
    hf = _combine(acc0_ref, acc1_ref, s0_ref, s1_ref, as_ref, ad_ref, h_ref,
                  b_ref)
    ps_ref[...] = jnp.dot(hf, w1a_ref[...], preferred_element_type=jnp.float32)
    pd_ref[...] = (
        jnp.dot(hf, w1b_ref[...], preferred_element_type=jnp.float32)
        + fb_ref[...]
    )


def _tc_combine_call(body, extra_specs, out_specs, out_shape, args):
    return pl.pallas_call(
        body,
        grid=(NP // NB,),
        in_specs=[
            pl.BlockSpec((NB, H), lambda i: (i, 0)),
            pl.BlockSpec((NB, H), lambda i: (i, 0)),
            pl.BlockSpec((NB,), lambda i: (i,)),
            pl.BlockSpec((NB,), lambda i: (i,)),
            pl.BlockSpec((NB,), lambda i: (i,)),
            pl.BlockSpec((NB,), lambda i: (i,)),
            pl.BlockSpec((NB, H), lambda i: (i, 0)),
            pl.BlockSpec((1, H), lambda i: (0, 0)),
        ] + extra_specs,
        out_specs=out_specs,
        out_shape=out_shape,
    )(*args)


def _tc_out_body(z_ref, bd_ref, bd1_ref, b_ref, o_ref):
    y = jnp.dot(z_ref[...], bd_ref[...], preferred_element_type=jnp.float32)
    y = y + b_ref[...]
    ey = jnp.exp(y)
    ssum = jnp.dot(ey, bd1_ref[...], preferred_element_type=jnp.float32)
    o_ref[...] = y - jnp.log(ssum)


def _tc_out(z2, bd, bd1, btile):
    return pl.pallas_call(
        _tc_out_body,
        grid=(ER // EB,),
        in_specs=[
            pl.BlockSpec((EB, 128), lambda i: (i, 0)),
            pl.BlockSpec((128, 128), lambda i: (0, 0)),
            pl.BlockSpec((128, 128), lambda i: (0, 0)),
            pl.BlockSpec((1, 128), lambda i: (0, 0)),
        ],
        out_specs=pl.BlockSpec((EB, 128), lambda i: (i, 0)),
        out_shape=jax.ShapeDtypeStruct((ER, 128), jnp.float32),
    )(z2, bd, bd1, btile)


def _sc_agg_body(srcdst, a_s, a_d, htab, acc0, acc1, s0, s1,
                 src_v, dst_v, as_v, ad_v,
                 hb0, hb1, hb2, exb0, exb1, exb2,
                 zrow, zcol, acc_sh, s_sh,
                 gs0, gs1, gs2, ss0, ss1, ss2):
    c = lax.axis_index("c")
    s = lax.axis_index("s")
    t = c * 16 + s
    nr = BR + (t < XT).astype(jnp.int32)
    roff = BR * t + jnp.minimum(t, XT)
    zv = jnp.zeros((16,), jnp.float32)
    hbufs = (hb0, hb1, hb2)
    exbufs = (exb0, exb1, exb2)
    gsems = (gs0, gs1, gs2)
    ssems = (ss0, ss1, ss2)

    def zfill(i, carry):
        zrow[i] = zv
        return carry

    lax.fori_loop(0, NPB, zfill, 0)

    def zfill1(i, carry):
        zcol[pl.ds(i * 16, 16)] = zv
        return carry

    lax.fori_loop(0, NPB // 16, zfill1, 0)
    pltpu.sync_copy(zrow, acc_sh.at[pl.ds(s * NPB, NPB)])
    pltpu.sync_copy(zcol, s_sh.at[pl.ds(s * NPB, NPB)])
    pltpu.sync_copy(a_s, as_v)
    pltpu.sync_copy(a_d, ad_v)
    pltpu.sync_copy(srcdst.at[0, pl.ds(roff * CH, BR * CH)],
                    src_v.at[pl.ds(0, BR * CH)])
    pltpu.sync_copy(srcdst.at[1, pl.ds(roff * CH, BR * CH)],
                    dst_v.at[pl.ds(0, BR * CH)])

    @pl.when(t < XT)
    def _():
        pltpu.sync_copy(srcdst.at[0, pl.ds((roff + BR) * CH, CH)],
                        src_v.at[pl.ds(BR * CH, CH)])
        pltpu.sync_copy(srcdst.at[1, pl.ds((roff + BR) * CH, CH)],
                        dst_v.at[pl.ds(BR * CH, CH)])

    plsc.subcore_barrier()

    def gissue(j, b):
        pltpu.async_copy(htab.at[src_v.at[pl.ds(j * CH, CH)]], hbufs[b],
                         gsems[b])

    def swait(b):
        dummy = dst_v.at[pl.ds(0, CH)]
        pltpu.make_async_copy(hbufs[b], acc_sh.at[dummy], ssems[b]).wait()
        pltpu.make_async_copy(exbufs[b], s_sh.at[dummy], ssems[b]).wait()

    def section(j, b):
        hb = hbufs[b]
        exb = exbufs[b]
        pltpu.make_async_copy(htab.at[src_v.at[pl.ds(j * CH, CH)]], hb,
                              gsems[b]).wait()
        for v in range(CH // 16):
            gsl = pl.ds(j * CH + v * 16, 16)
            sidx = src_v[gsl]
            didx = dst_v[gsl]
            av = plsc.load_gather(as_v, [sidx])
            bv = plsc.load_gather(ad_v, [didx])
            e = av + bv
            e = jnp.where(e >= 0.0, e, 0.2 * e)
            ex = jnp.exp(e)
            exb[pl.ds(v * 16, 16)] = ex
            for l in range(16):
                k = v * 16 + l
                hb[k] = hb[k] * ex[l]
        didx_row = dst_v.at[pl.ds(j * CH, CH)]
        pltpu.async_copy(hb, acc_sh.at[didx_row], ssems[b], add=True)
        pltpu.async_copy(exb, s_sh.at[didx_row], ssems[b], add=True)
        nb = (b + 2) % 3

        @pl.when(j < nr - 2)
        def _():
            @pl.when(j >= 1)
            def _():
                swait(nb)

            gissue(j + 2, nb)

    gissue(0, 0)
    gissue(1, 1)

    def gbody(g, carry):
        for k in range(3):
            section(3 * g + k, k)
        return carry

    lax.fori_loop(0, GR, gbody, 0)

    @pl.when(nr > BR)
    def _():
        section(BR, BR % 3)

    swait(0)
    swait(1)
    swait(2)
    plsc.subcore_barrier()

    @pl.when(c == 0)
    def _():
        pltpu.sync_copy(acc_sh.at[pl.ds(s * NPB, NPB)], acc0.at[pl.ds(s * NPB, NPB)])
        pltpu.sync_copy(s_sh.at[pl.ds(s * NPB, NPB)], s0.at[pl.ds(s * NPB, NPB)])

    @pl.when(c == 1)
    def _():
        pltpu.sync_copy(acc_sh.at[pl.ds(s * NPB, NPB)], acc1.at[pl.ds(s * NPB, NPB)])
        pltpu.sync_copy(s_sh.at[pl.ds(s * NPB, NPB)], s1.at[pl.ds(s * NPB, NPB)])


_sc_agg = functools.partial(
    pl.kernel,
    out_type=[
        jax.ShapeDtypeStruct((NP, H), jnp.float32),
        jax.ShapeDtypeStruct((NP, H), jnp.float32),
        jax.ShapeDtypeStruct((NP,), jnp.float32),
        jax.ShapeDtypeStruct((NP,), jnp.float32),
    ],
    mesh=_MESH,
    compiler_params=_SC_PARAMS,
    scratch_types=[
        pltpu.VMEM(((BR + 3) * CH,), jnp.int32),
        pltpu.VMEM(((BR + 3) * CH,), jnp.int32),
        pltpu.VMEM((NP,), jnp.float32),
        pltpu.VMEM((NP,), jnp.float32),
        pltpu.VMEM((CH, H), jnp.float32),
        pltpu.VMEM((CH, H), jnp.float32),
        pltpu.VMEM((CH, H), jnp.float32),
        pltpu.VMEM((CH,), jnp.float32),
        pltpu.VMEM((CH,), jnp.float32),
        pltpu.VMEM((CH,), jnp.float32),
        pltpu.VMEM((NPB, H), jnp.float32),
        pltpu.VMEM((NPB,), jnp.float32),
        pltpu.VMEM_SHARED((NP, H), jnp.float32),
        pltpu.VMEM_SHARED((NP,), jnp.float32),
        pltpu.SemaphoreType.DMA,
        pltpu.SemaphoreType.DMA,
        pltpu.SemaphoreType.DMA,
        pltpu.SemaphoreType.DMA,
        pltpu.SemaphoreType.DMA,
        pltpu.SemaphoreType.DMA,
    ],
)(_sc_agg_body)


def _sc_mlp_body(srcdst, ps_tab, pd_tab, z_out,
                 src_v, dst_v, pa0, pa1, pa2, pb0, pb1, pb2,
                 wb0, wb1, wb2, gs0, gs1, gs2, ws0, ws1, ws2):
    c = lax.axis_index("c")
    s = lax.axis_index("s")
    t = c * 16 + s
    nr = BR + (t < XT).astype(jnp.int32)
    roff = BR * t + jnp.minimum(t, XT)
    zbase = roff * (CH // 8)
    pas = (pa0, pa1, pa2)
    pbs = (pb0, pb1, pb2)
    wbs = (wb0, wb1, wb2)
    gsems = (gs0, gs1, gs2)
    wsems = (ws0, ws1, ws2)
    pltpu.sync_copy(srcdst.at[0, pl.ds(roff * CH, BR * CH)],
                    src_v.at[pl.ds(0, BR * CH)])
    pltpu.sync_copy(srcdst.at[1, pl.ds(roff * CH, BR * CH)],
                    dst_v.at[pl.ds(0, BR * CH)])

    @pl.when(t < XT)
    def _():
        pltpu.sync_copy(srcdst.at[0, pl.ds((roff + BR) * CH, CH)],
                        src_v.at[pl.ds(BR * CH, CH)])
        pltpu.sync_copy(srcdst.at[1, pl.ds((roff + BR) * CH, CH)],
                        dst_v.at[pl.ds(BR * CH, CH)])

    def gissue(j, b):
        pltpu.async_copy(ps_tab.at[src_v.at[pl.ds(j * CH, CH)]], pas[b],
                         gsems[b])
        pltpu.async_copy(pd_tab.at[dst_v.at[pl.ds(j * CH, CH)]], pbs[b],
                         gsems[b])

    def wwait(b):
        pltpu.make_async_copy(wbs[b], z_out.at[pl.ds(0, CH // 8)], wsems[b]).wait()

    def section(j, b):
        pa = pas[b]
        pb = pbs[b]
        wb = wbs[b]
        pltpu.make_async_copy(ps_tab.at[src_v.at[pl.ds(j * CH, CH)]], pa,
                              gsems[b]).wait()
        pltpu.make_async_copy(pd_tab.at[dst_v.at[pl.ds(j * CH, CH)]], pb,
                              gsems[b]).wait()
        for k in range(CH):
            wb[k // 8, pl.ds((k % 8) * H, H)] = jnp.maximum(pa[k] + pb[k], 0.0)
        pltpu.async_copy(wb, z_out.at[pl.ds(zbase + j * (CH // 8), CH // 8)], wsems[b])
        nb = (b + 2) % 3

        @pl.when(j < nr - 2)
        def _():
            @pl.when(j >= 1)
            def _():
                wwait(nb)

            gissue(j + 2, nb)

    gissue(0, 0)
    gissue(1, 1)

    def gbody(g, carry):
        for k in range(3):
            section(3 * g + k, k)
        return carry

    lax.fori_loop(0, GR, gbody, 0)

    @pl.when(nr > BR)
    def _():
        section(BR, BR % 3)

    wwait(0)
    wwait(1)
    wwait(2)


_sc_mlp = functools.partial(
    pl.kernel,
    out_type=jax.ShapeDtypeStruct((ER, 128), jnp.float32),
    mesh=_MESH,
    compiler_params=_SC_PARAMS,
    scratch_types=[
        pltpu.VMEM(((BR + 3) * CH,), jnp.int32),
        pltpu.VMEM(((BR + 3) * CH,), jnp.int32),
        pltpu.VMEM((CH, H), jnp.float32),
        pltpu.VMEM((CH, H), jnp.float32),
        pltpu.VMEM((CH, H), jnp.float32),
        pltpu.VMEM((CH, H), jnp.float32),
        pltpu.VMEM((CH, H), jnp.float32),
        pltpu.VMEM((CH, H), jnp.float32),
        pltpu.VMEM((CH // 8, 128), jnp.float32),
        pltpu.VMEM((CH // 8, 128), jnp.float32),
        pltpu.VMEM((CH // 8, 128), jnp.float32),
        pltpu.SemaphoreType.DMA,
        pltpu.SemaphoreType.DMA,
        pltpu.SemaphoreType.DMA,
        pltpu.SemaphoreType.DMA,
        pltpu.SemaphoreType.DMA,
        pltpu.SemaphoreType.DMA,
    ],
)(_sc_mlp_body)


def kernel(x, edge_index, W1, a_src1, a_dst1, b1, W2, a_src2, a_dst2, b2,
           fc1_W, fc1_b, fc2_W, fc2_b):
    srcdst = edge_index.astype(jnp.int32)
    wa1 = jnp.concatenate(
        [W1.T, a_src1[:, None], a_dst1[:, None]], axis=1)
    A2 = jnp.stack([a_src2, a_dst2], axis=1)

    h1, as1, ad1 = _tc_front(x, wa1)
    acc0, acc1, s0, s1 = _sc_agg(srcdst, as1, ad1, h1)
    h2, as2, ad2 = _tc_combine_call(
        _tc_mid_body,
        [pl.BlockSpec((H, H), lambda i: (0, 0)),
         pl.BlockSpec((H, 2), lambda i: (0, 0))],
        [pl.BlockSpec((NB, H), lambda i: (i, 0)),
         pl.BlockSpec((NB,), lambda i: (i,)),
         pl.BlockSpec((NB,), lambda i: (i,))],
        [jax.ShapeDtypeStruct((NP, H), jnp.float32),
         jax.ShapeDtypeStruct((NP,), jnp.float32),
         jax.ShapeDtypeStruct((NP,), jnp.float32)],
        (acc0, acc1, s0, s1, as1, ad1, h1, b1[None, :], W2, A2),
    )
    acc0b, acc1b, s0b, s1b = _sc_agg(srcdst, as2, ad2, h2)
    ps, pd = _tc_combine_call(
        _tc_tail_body,
        [pl.BlockSpec((H, H), lambda i: (0, 0)),
         pl.BlockSpec((H, H), lambda i: (0, 0)),
         pl.BlockSpec((1, H), lambda i: (0, 0))],
        [pl.BlockSpec((NB, H), lambda i: (i, 0)),
         pl.BlockSpec((NB, H), lambda i: (i, 0))],
        [jax.ShapeDtypeStruct((NP, H), jnp.float32),
         jax.ShapeDtypeStruct((NP, H), jnp.float32)],
        (acc0b, acc1b, s0b, s1b, as2, ad2, h2, b2[None, :],
         fc1_W[:H], fc1_W[H:], fc1_b[None, :]),
    )
    z2 = _sc_mlp(srcdst, ps, pd)
    eye8 = jnp.eye(8, dtype=jnp.float32)
    bd = jnp.kron(eye8, fc2_W)
    bd1 = jnp.kron(eye8, jnp.ones((H, NC), jnp.float32))
    btile = jnp.tile(fc2_b, 8)[None, :]
    out2 = _tc_out(z2, bd, bd1, btile)
    return out2.reshape(E, NC)
